# baseline (speedup 1.0000x reference)
import functools

import jax
import jax.numpy as jnp
from jax import lax
from jax.experimental import pallas as pl
from jax.experimental.pallas import tpu as pltpu
from jax.experimental.pallas import tpu_sc as plsc

NUM_EMB = 1000000
D = 32
S = 50
NB = 16384
LANE = 128
DT = D // 8

_info = plsc.get_sparse_core_info()
NC = _info.num_cores
NS = _info.num_subcores
NW = NC * NS

BLK = 512
NCHUNK = S * NB // BLK // NW
CPG = NB // LANE // 4


def _iota16():
    return lax.iota(jnp.int32, 16)


@functools.partial(
    pl.kernel,
    out_type=jax.ShapeDtypeStruct((S, DT, NB // LANE, 8, LANE), jnp.float32),
    mesh=plsc.VectorSubcoreMesh(core_axis_name="c", subcore_axis_name="s"),
    scratch_types=[
        pltpu.VMEM((2, 4, LANE), jnp.int32),
        pltpu.VMEM((2, BLK, D), jnp.float32),
        pltpu.VMEM((2, DT, 4, 8, LANE), jnp.float32),
        [pltpu.SemaphoreType.DMA] * 2,
        [pltpu.SemaphoreType.DMA] * 2,
        [pltpu.SemaphoreType.DMA] * 2,
    ],
    compiler_params=pltpu.CompilerParams(
        use_tc_tiling_on_sc=False, needs_layout_passes=False),
)
def _emb_lookup(weight_hbm, xt_hbm, out_hbm, idx_v, rows_v, tile_v,
                isems, gsems, ssems):
    wid = lax.axis_index("s") * NC + lax.axis_index("c")
    g0 = wid * NCHUNK

    def sblk(c):
        g = g0 + c
        return g // CPG, lax.rem(g, CPG)

    def fire_idx(c, p):
        s, blk = sblk(c)
        pltpu.async_copy(xt_hbm.at[s, pl.ds(blk * 4, 4)], idx_v.at[p], isems[p])

    def wait_idx(c, p):
        s, blk = sblk(c)
        pltpu.make_async_copy(
            xt_hbm.at[s, pl.ds(blk * 4, 4)], idx_v.at[p], isems[p]).wait()

    def fire_gathers(p):
        for j in range(4):
            pltpu.async_copy(
                weight_hbm.at[idx_v.at[p, j]],
                rows_v.at[p, pl.ds(j * LANE, LANE)],
                gsems[p],
            )

    def wait_gathers(p):
        for j in range(4):
            pltpu.make_async_copy(
                weight_hbm.at[idx_v.at[p, j]],
                rows_v.at[p, pl.ds(j * LANE, LANE)],
                gsems[p],
            ).wait()

    def fire_scatters(c, p):
        s, blk = sblk(c)
        for dt in range(DT):
            pltpu.async_copy(
                tile_v.at[p, dt],
                out_hbm.at[s, dt, pl.ds(blk * 4, 4)],
                ssems[p],
            )

    def wait_scatters(c, p):
        s, blk = sblk(c)
        for dt in range(DT):
            pltpu.make_async_copy(
                tile_v.at[p, dt],
                out_hbm.at[s, dt, pl.ds(blk * 4, 4)],
                ssems[p],
            ).wait()

    def transpose_chunk(p):
        rows = rows_v.at[p]
        iota = _iota16()
        dsplat = [jnp.full((16,), d, jnp.int32) for d in range(D)]

        def loads(g, lo_d, hi_d):
            bidx = g * 16 + iota
            return [plsc.load_gather(rows, [bidx, dsplat[d]])
                    for d in range(lo_d, hi_d)]

        def stores(g, lo_d, vs):
            btl = g // 8
            lo = lax.rem(g, 8) * 16
            for k, v in enumerate(vs):
                d = lo_d + k
                tile_v[p, d // 8, btl, d % 8, pl.ds(lo, 16)] = v

        vs0 = loads(0, 0, D)

        def tbody(i, vs):
            n_lo = loads(i + 1, 0, D // 2)
            stores(i, 0, vs[:D // 2])
            n_hi = loads(i + 1, D // 2, D)
            stores(i, D // 2, vs[D // 2:])
            return n_lo + n_hi

        vs_last = lax.fori_loop(0, BLK // 16 - 1, tbody, vs0)
        stores(BLK // 16 - 1, 0, vs_last)

    def body(c, p, first, last):
        if not last:
            fire_idx(c + 1, 1 - p)
        wait_gathers(p)
        if not first:
            wait_scatters(c - 2, p)
        if not last:
            wait_idx(c + 1, 1 - p)
            fire_gathers(1 - p)
        transpose_chunk(p)
        fire_scatters(c, p)

    fire_idx(0, 0)
    wait_idx(0, 0)
    fire_gathers(0)

    body(0, 0, True, False)
    body(1, 1, True, False)

    def group(g, _):
        c = 2 + 2 * g
        body(c, 0, False, False)
        body(c + 1, 1, False, False)
        return 0

    lax.fori_loop(0, (NCHUNK - 4) // 2, group, 0)

    body(NCHUNK - 2, 0, False, False)
    body(NCHUNK - 1, 1, False, True)

    wait_scatters(NCHUNK - 2, 0)
    wait_scatters(NCHUNK - 1, 1)


def kernel(x, weight):
    xt = x.T.reshape(S, NB // LANE, LANE)
    phys = _emb_lookup(weight, xt)
    return phys.transpose(2, 4, 0, 1, 3).reshape(NB, S, D)


# speedup vs baseline: 1.0090x; 1.0090x over previous
import functools

import jax
import jax.numpy as jnp
from jax import lax
from jax.experimental import pallas as pl
from jax.experimental.pallas import tpu as pltpu
from jax.experimental.pallas import tpu_sc as plsc

NUM_EMB = 1000000
D = 32
S = 50
NB = 16384
LANE = 128
DT = D // 8

_info = plsc.get_sparse_core_info()
NC = _info.num_cores
NS = _info.num_subcores
NW = NC * NS

BLK = 512
NCHUNK = S * NB // BLK // NW
CPG = NB // LANE // 4


def _iota16():
    return lax.iota(jnp.int32, 16)


@functools.partial(
    pl.kernel,
    out_type=jax.ShapeDtypeStruct((S, DT, NB // LANE, 8, LANE), jnp.float32),
    mesh=plsc.VectorSubcoreMesh(core_axis_name="c", subcore_axis_name="s"),
    scratch_types=[
        pltpu.VMEM((2, 4, LANE), jnp.int32),
        pltpu.VMEM((2, BLK, D), jnp.float32),
        pltpu.VMEM((2, DT, 4, 8, LANE), jnp.float32),
        [pltpu.SemaphoreType.DMA] * 2,
        [pltpu.SemaphoreType.DMA] * 2,
        [pltpu.SemaphoreType.DMA] * 2,
    ],
    compiler_params=pltpu.CompilerParams(
        use_tc_tiling_on_sc=False, needs_layout_passes=False),
)
def _emb_lookup(weight_hbm, xt_hbm, out_hbm, idx_v, rows_v, tile_v,
                isems, gsems, ssems):
    wid = lax.axis_index("s") * NC + lax.axis_index("c")
    g0 = wid * NCHUNK

    def sblk(c):
        g = g0 + c
        return g // CPG, lax.rem(g, CPG)

    def fire_idx(c, p):
        s, blk = sblk(c)
        pltpu.async_copy(xt_hbm.at[s, pl.ds(blk * 4, 4)], idx_v.at[p], isems[p])

    def wait_idx(c, p):
        s, blk = sblk(c)
        pltpu.make_async_copy(
            xt_hbm.at[s, pl.ds(blk * 4, 4)], idx_v.at[p], isems[p]).wait()

    def fire_gathers(p):
        for j in range(4):
            pltpu.async_copy(
                weight_hbm.at[idx_v.at[p, j]],
                rows_v.at[p, pl.ds(j * LANE, LANE)],
                gsems[p],
            )

    def wait_gathers(p):
        for j in range(4):
            pltpu.make_async_copy(
                weight_hbm.at[idx_v.at[p, j]],
                rows_v.at[p, pl.ds(j * LANE, LANE)],
                gsems[p],
            ).wait()

    def fire_scatters(c, p):
        s, blk = sblk(c)
        for dt in range(DT):
            pltpu.async_copy(
                tile_v.at[p, dt],
                out_hbm.at[s, dt, pl.ds(blk * 4, 4)],
                ssems[p],
            )

    def wait_scatters(c, p):
        s, blk = sblk(c)
        for dt in range(DT):
            pltpu.make_async_copy(
                tile_v.at[p, dt],
                out_hbm.at[s, dt, pl.ds(blk * 4, 4)],
                ssems[p],
            ).wait()

    def transpose_chunk(p):
        rows = rows_v.at[p]
        iota = _iota16()
        dsplat = [jnp.full((16,), d, jnp.int32) for d in range(D)]

        bvec = [gi * 16 + iota for gi in range(8)]

        def tbody(btl, _):
            boff = btl * LANE
            for gi in range(8):
                bidx = boff + bvec[gi]
                vs = [plsc.load_gather(rows, [bidx, dsplat[d]]) for d in range(D)]
                for d in range(D):
                    tile_v[p, d // 8, btl, d % 8, pl.ds(gi * 16, 16)] = vs[d]
            return 0

        lax.fori_loop(0, BLK // LANE, tbody, 0)

    def body(c, p, first, last):
        if not last:
            fire_idx(c + 1, 1 - p)
        wait_gathers(p)
        if not first:
            wait_scatters(c - 2, p)
        if not last:
            wait_idx(c + 1, 1 - p)
            fire_gathers(1 - p)
        transpose_chunk(p)
        fire_scatters(c, p)

    fire_idx(0, 0)
    wait_idx(0, 0)
    fire_gathers(0)

    body(0, 0, True, False)
    body(1, 1, True, False)

    def group(g, _):
        c = 2 + 2 * g
        body(c, 0, False, False)
        body(c + 1, 1, False, False)
        return 0

    lax.fori_loop(0, (NCHUNK - 4) // 2, group, 0)

    body(NCHUNK - 2, 0, False, False)
    body(NCHUNK - 1, 1, False, True)

    wait_scatters(NCHUNK - 2, 0)
    wait_scatters(NCHUNK - 1, 1)


def kernel(x, weight):
    xt = x.T.reshape(S, NB // LANE, LANE)
    phys = _emb_lookup(weight, xt)
    return phys.transpose(2, 4, 0, 1, 3).reshape(NB, S, D)
